# baseline (speedup 1.0000x reference)
import functools

import jax
import jax.numpy as jnp
from jax import lax
from jax.experimental import pallas as pl
from jax.experimental.pallas import tpu as pltpu
from jax.experimental.pallas import tpu_sc as plsc

N = 10000
F = 128
H = 16
E = 320000

NC, NS = 2, 16
NW = NC * NS
IDXW = 128
ROWS = E // IDXW
RPW = ROWS // NW
XT = ROWS - RPW * NW
KJ = 6
NSS = RPW // KJ
CH = KJ * IDXW
ACC_ROWS = 10240
RPT = ACC_ROWS // NS

_mesh = plsc.VectorSubcoreMesh(core_axis_name="c", subcore_axis_name="s")


@functools.partial(
    pl.kernel,
    mesh=_mesh,
    out_type=jax.ShapeDtypeStruct((NC, ACC_ROWS, H), jnp.float32),
    scratch_types=[
        pltpu.VMEM((RPW + 1, IDXW), jnp.int32),
        pltpu.VMEM((RPW + 1, IDXW), jnp.int32),
        pltpu.VMEM((CH, H), jnp.float32),
        pltpu.VMEM((CH, H), jnp.float32),
        pltpu.VMEM((RPT, H), jnp.float32),
        pltpu.VMEM_SHARED((ACC_ROWS, H), jnp.float32),
        pltpu.SemaphoreType.DMA,
        pltpu.SemaphoreType.DMA,
    ],
    compiler_params=pltpu.CompilerParams(use_tc_tiling_on_sc=False),
)
def _agg_sc(m_hbm, src_hbm, dst_hbm, out_hbm, sidx, didx, rows0, rows1, zbuf, acc, sem0, sem1):
    c = lax.axis_index("c")
    s = lax.axis_index("s")
    wid = c * NS + s
    has_xtra = wid < XT
    start = wid * RPW + jnp.minimum(wid, XT)
    rowsb = (rows0, rows1)
    sems = (sem0, sem1)

    pltpu.sync_copy(src_hbm.at[pl.ds(start, RPW)], sidx.at[pl.ds(0, RPW)])
    pltpu.sync_copy(dst_hbm.at[pl.ds(start, RPW)], didx.at[pl.ds(0, RPW)])

    @pl.when(has_xtra)
    def _():
        pltpu.sync_copy(src_hbm.at[pl.ds(start + RPW, 1)], sidx.at[pl.ds(RPW, 1)])
        pltpu.sync_copy(dst_hbm.at[pl.ds(start + RPW, 1)], didx.at[pl.ds(RPW, 1)])

    def fire(ss):
        buf = rowsb[ss % 2]
        return [
            pltpu.async_copy(
                m_hbm.at[sidx.at[ss * KJ + j]],
                buf.at[pl.ds(j * IDXW, IDXW)],
                sems[ss % 2],
            )
            for j in range(KJ)
        ]

    pend = {0: fire(0), 1: fire(1)}

    def _z(i, carry):
        zbuf[i, :] = jnp.zeros((H,), jnp.float32)
        return carry

    lax.fori_loop(0, RPT, _z, 0)
    pltpu.sync_copy(zbuf, acc.at[pl.ds(s * RPT, RPT)])
    plsc.subcore_barrier()

    for ss in range(NSS):
        p = ss % 2
        for cp in pend.pop(ss):
            cp.wait()
        buf = rowsb[p]
        for j in range(KJ):
            pltpu.sync_copy(
                buf.at[pl.ds(j * IDXW, IDXW)], acc.at[didx.at[ss * KJ + j]], add=True
            )
        if ss + 2 < NSS:
            pend[ss + 2] = fire(ss + 2)

    @pl.when(has_xtra)
    def _():
        pltpu.async_copy(
            m_hbm.at[sidx.at[RPW]], rows0.at[pl.ds(0, IDXW)], sem0
        ).wait()
        pltpu.sync_copy(rows0.at[pl.ds(0, IDXW)], acc.at[didx.at[RPW]], add=True)

    plsc.subcore_barrier()

    pltpu.sync_copy(acc.at[pl.ds(s * RPT, RPT)], zbuf)
    pltpu.sync_copy(zbuf, out_hbm.at[c].at[pl.ds(s * RPT, RPT)])


@functools.partial(
    pl.kernel,
    mesh=_mesh,
    out_type=jax.ShapeDtypeStruct((NC, ACC_ROWS, H), jnp.float32),
    scratch_types=[
        pltpu.VMEM((RPW + 1, IDXW), jnp.int32),
        pltpu.VMEM((RPT, H), jnp.float32),
        pltpu.VMEM_SHARED((ACC_ROWS, H), jnp.float32),
    ],
    compiler_params=pltpu.CompilerParams(use_tc_tiling_on_sc=False),
)
def _deg_sc(dst_hbm, out_hbm, didx, rows, acc):
    c = lax.axis_index("c")
    s = lax.axis_index("s")
    wid = c * NS + s
    has_xtra = wid < XT
    start = wid * RPW + jnp.minimum(wid, XT)

    pltpu.sync_copy(dst_hbm.at[pl.ds(start, RPW)], didx.at[pl.ds(0, RPW)])

    @pl.when(has_xtra)
    def _():
        pltpu.sync_copy(dst_hbm.at[pl.ds(start + RPW, 1)], didx.at[pl.ds(RPW, 1)])

    def _z(i, carry):
        rows[i, :] = jnp.zeros((H,), jnp.float32)
        return carry

    lax.fori_loop(0, RPT, _z, 0)
    pltpu.sync_copy(rows, acc.at[pl.ds(s * RPT, RPT)])
    plsc.subcore_barrier()

    def _o(i, carry):
        rows[i, :] = jnp.ones((H,), jnp.float32)
        return carry

    lax.fori_loop(0, IDXW, _o, 0)

    def _step(r, carry):
        pltpu.sync_copy(rows.at[pl.ds(0, IDXW)], acc.at[didx.at[r]], add=True)
        return carry

    lax.fori_loop(0, RPW, _step, 0)

    @pl.when(has_xtra)
    def _():
        pltpu.sync_copy(rows.at[pl.ds(0, IDXW)], acc.at[didx.at[RPW]], add=True)

    plsc.subcore_barrier()

    pltpu.sync_copy(acc.at[pl.ds(s * RPT, RPT)], rows)
    pltpu.sync_copy(rows, out_hbm.at[c].at[pl.ds(s * RPT, RPT)])


def _tc1_body(degp_ref, x_ref, w1_ref, m1_ref, dmat_ref):
    deg = degp_ref[0, :N, :] + degp_ref[1, :N, :] + 1.0
    d = lax.rsqrt(deg)
    u1 = jnp.dot(x_ref[...], w1_ref[...], preferred_element_type=jnp.float32)
    m1_ref[...] = d * u1
    dmat_ref[...] = d


def _tc2_body(p1_ref, m1_ref, dmat_ref, b1_ref, w2_ref, m2_ref):
    d = dmat_ref[...]
    h = d * (p1_ref[0, :N, :] + p1_ref[1, :N, :] + m1_ref[...]) + b1_ref[...]
    h = jnp.maximum(h, 0.0)
    m2_ref[...] = d * jnp.dot(h, w2_ref[...], preferred_element_type=jnp.float32)


def _tc3_body(p2_ref, m2_ref, dmat_ref, b2_ref, wl_ref, bl_ref, out_ref):
    d = dmat_ref[...]
    h = d * (p2_ref[0, :N, :] + p2_ref[1, :N, :] + m2_ref[...]) + b2_ref[...]
    h = jnp.maximum(h, 0.0)
    out_ref[...] = (
        jnp.dot(h, wl_ref[...], preferred_element_type=jnp.float32) + bl_ref[...]
    )


def kernel(x, edge_index, W1, b1, W2, b2, Wl, bl):
    src2d = edge_index[0].reshape(ROWS, IDXW)
    dst2d = edge_index[1].reshape(ROWS, IDXW)

    degp = _deg_sc(dst2d)

    m1, dmat = pl.pallas_call(
        _tc1_body,
        out_shape=(
            jax.ShapeDtypeStruct((N, H), jnp.float32),
            jax.ShapeDtypeStruct((N, H), jnp.float32),
        ),
    )(degp, x, W1)

    p1 = _agg_sc(m1, src2d, dst2d)

    m2 = pl.pallas_call(
        _tc2_body,
        out_shape=jax.ShapeDtypeStruct((N, H), jnp.float32),
    )(p1, m1, dmat, b1.reshape(1, H), W2)

    p2 = _agg_sc(m2, src2d, dst2d)

    out = pl.pallas_call(
        _tc3_body,
        out_shape=jax.ShapeDtypeStruct((N, 1), jnp.float32),
    )(p2, m2, dmat, b2.reshape(1, H), Wl, bl.reshape(1, 1))

    return out.reshape(-1)


# speedup vs baseline: 1.4475x; 1.4475x over previous
import functools

import jax
import jax.numpy as jnp
from jax import lax
from jax.experimental import pallas as pl
from jax.experimental.pallas import tpu as pltpu
from jax.experimental.pallas import tpu_sc as plsc

N = 10000
F = 128
H = 16
E = 320000

NC, NS = 2, 16
NW = NC * NS
IDXW = 128
ROWS = E // IDXW
RPW = ROWS // NW
XT = ROWS - RPW * NW
KJ = 6
NSS = RPW // KJ
CH = KJ * IDXW
ACC_ROWS = 10240
RPT = ACC_ROWS // NS

_mesh = plsc.VectorSubcoreMesh(core_axis_name="c", subcore_axis_name="s")


@functools.partial(
    pl.kernel,
    mesh=_mesh,
    out_type=jax.ShapeDtypeStruct((NC, ACC_ROWS, H), jnp.float32),
    scratch_types=[
        pltpu.VMEM((RPW + 1, IDXW), jnp.int32),
        pltpu.VMEM((RPW + 1, IDXW), jnp.int32),
        pltpu.VMEM((CH, H), jnp.float32),
        pltpu.VMEM((CH, H), jnp.float32),
        pltpu.VMEM((RPT, H), jnp.float32),
        pltpu.VMEM_SHARED((ACC_ROWS, H), jnp.float32),
        pltpu.SemaphoreType.DMA,
        pltpu.SemaphoreType.DMA,
    ],
    compiler_params=pltpu.CompilerParams(use_tc_tiling_on_sc=False),
)
def _agg_sc(m_hbm, src_hbm, dst_hbm, out_hbm, sidx, didx, rows0, rows1, zbuf, acc, sem0, sem1):
    c = lax.axis_index("c")
    s = lax.axis_index("s")
    wid = c * NS + s
    has_xtra = wid < XT
    start = wid * RPW + jnp.minimum(wid, XT)
    rowsb = (rows0, rows1)
    sems = (sem0, sem1)

    pltpu.sync_copy(src_hbm.at[pl.ds(start, RPW)], sidx.at[pl.ds(0, RPW)])
    pltpu.sync_copy(dst_hbm.at[pl.ds(start, RPW)], didx.at[pl.ds(0, RPW)])

    @pl.when(has_xtra)
    def _():
        pltpu.sync_copy(src_hbm.at[pl.ds(start + RPW, 1)], sidx.at[pl.ds(RPW, 1)])
        pltpu.sync_copy(dst_hbm.at[pl.ds(start + RPW, 1)], didx.at[pl.ds(RPW, 1)])

    def fire(ss):
        buf = rowsb[ss % 2]
        return [
            pltpu.async_copy(
                m_hbm.at[sidx.at[ss * KJ + j]],
                buf.at[pl.ds(j * IDXW, IDXW)],
                sems[ss % 2],
            )
            for j in range(KJ)
        ]

    pend = {0: fire(0), 1: fire(1)}

    def _z(i, carry):
        zbuf[i, :] = jnp.zeros((H,), jnp.float32)
        return carry

    lax.fori_loop(0, RPT, _z, 0)
    pltpu.sync_copy(zbuf, acc.at[pl.ds(s * RPT, RPT)])
    plsc.subcore_barrier()

    for ss in range(NSS):
        p = ss % 2
        for cp in pend.pop(ss):
            cp.wait()
        buf = rowsb[p]
        for j in range(KJ):
            pltpu.sync_copy(
                buf.at[pl.ds(j * IDXW, IDXW)], acc.at[didx.at[ss * KJ + j]], add=True
            )
        if ss + 2 < NSS:
            pend[ss + 2] = fire(ss + 2)

    @pl.when(has_xtra)
    def _():
        pltpu.async_copy(
            m_hbm.at[sidx.at[RPW]], rows0.at[pl.ds(0, IDXW)], sem0
        ).wait()
        pltpu.sync_copy(rows0.at[pl.ds(0, IDXW)], acc.at[didx.at[RPW]], add=True)

    plsc.subcore_barrier()

    pltpu.sync_copy(acc.at[pl.ds(s * RPT, RPT)], zbuf)
    pltpu.sync_copy(zbuf, out_hbm.at[c].at[pl.ds(s * RPT, RPT)])


@functools.partial(
    pl.kernel,
    mesh=_mesh,
    out_type=jax.ShapeDtypeStruct((NC, ACC_ROWS, H), jnp.float32),
    scratch_types=[
        pltpu.VMEM((RPW + 1, IDXW), jnp.int32),
        pltpu.VMEM((RPT, H), jnp.float32),
        pltpu.VMEM_SHARED((ACC_ROWS, H), jnp.float32),
    ],
    compiler_params=pltpu.CompilerParams(use_tc_tiling_on_sc=False),
)
def _deg_sc(dst_hbm, out_hbm, didx, rows, acc):
    c = lax.axis_index("c")
    s = lax.axis_index("s")
    wid = c * NS + s
    has_xtra = wid < XT
    start = wid * RPW + jnp.minimum(wid, XT)

    pltpu.sync_copy(dst_hbm.at[pl.ds(start, RPW)], didx.at[pl.ds(0, RPW)])

    @pl.when(has_xtra)
    def _():
        pltpu.sync_copy(dst_hbm.at[pl.ds(start + RPW, 1)], didx.at[pl.ds(RPW, 1)])

    def _z(i, carry):
        rows[i, :] = jnp.zeros((H,), jnp.float32)
        return carry

    lax.fori_loop(0, RPT, _z, 0)
    pltpu.sync_copy(rows, acc.at[pl.ds(s * RPT, RPT)])
    plsc.subcore_barrier()

    def _o(i, carry):
        rows[i, :] = jnp.ones((H,), jnp.float32)
        return carry

    lax.fori_loop(0, IDXW, _o, 0)

    def _step(r, carry):
        pltpu.sync_copy(rows.at[pl.ds(0, IDXW)], acc.at[didx.at[r]], add=True)
        return carry

    lax.fori_loop(0, RPW, _step, 0)

    @pl.when(has_xtra)
    def _():
        pltpu.sync_copy(rows.at[pl.ds(0, IDXW)], acc.at[didx.at[RPW]], add=True)

    plsc.subcore_barrier()

    pltpu.sync_copy(acc.at[pl.ds(s * RPT, RPT)], rows)
    pltpu.sync_copy(rows, out_hbm.at[c].at[pl.ds(s * RPT, RPT)])


NP8 = N // 8
AP8 = ACC_ROWS // 8


def _tc1_body(degp_ref, x3_ref, w1_ref, m1p_ref, dmatp_ref):
    dd = degp_ref[0] + degp_ref[1] + 1.0
    dp = lax.rsqrt(dd)
    dmatp_ref[...] = dp
    for j in range(8):
        u1j = jnp.dot(
            x3_ref[:, j, :], w1_ref[...], preferred_element_type=jnp.float32
        )
        m1p_ref[:, j * H : (j + 1) * H] = dp[:NP8, j * H : (j + 1) * H] * u1j


def _tc2_body(p1_ref, m1p_ref, dmatp_ref, b1t_ref, w2b_ref, m2p_ref):
    dp = dmatp_ref[:NP8, :]
    h = dp * (p1_ref[0, :NP8, :] + p1_ref[1, :NP8, :] + m1p_ref[...]) + b1t_ref[...]
    h = jnp.maximum(h, 0.0)
    m2p_ref[...] = dp * jnp.dot(h, w2b_ref[...], preferred_element_type=jnp.float32)


def _tc3_body(p2_ref, m2p_ref, dmatp_ref, b2t_ref, wls_ref, bl_ref, out_ref):
    dp = dmatp_ref[:NP8, :]
    h = dp * (p2_ref[0, :NP8, :] + p2_ref[1, :NP8, :] + m2p_ref[...]) + b2t_ref[...]
    h = jnp.maximum(h, 0.0)
    out_ref[...] = (
        jnp.dot(h, wls_ref[...], preferred_element_type=jnp.float32) + bl_ref[...]
    )


def kernel(x, edge_index, W1, b1, W2, b2, Wl, bl):
    src2d = edge_index[0].reshape(ROWS, IDXW)
    dst2d = edge_index[1].reshape(ROWS, IDXW)

    eye8 = jnp.eye(8, dtype=jnp.float32)
    w2b = jnp.kron(eye8, W2)
    wls = jnp.kron(eye8, Wl)
    b1t = jnp.tile(b1, 8).reshape(1, 128)
    b2t = jnp.tile(b2, 8).reshape(1, 128)

    degp = _deg_sc(dst2d)

    m1p, dmatp = pl.pallas_call(
        _tc1_body,
        out_shape=(
            jax.ShapeDtypeStruct((NP8, 128), jnp.float32),
            jax.ShapeDtypeStruct((AP8, 128), jnp.float32),
        ),
    )(degp.reshape(NC, AP8, 128), x.reshape(NP8, 8, F), W1)

    p1 = _agg_sc(m1p.reshape(N, H), src2d, dst2d)

    m2p = pl.pallas_call(
        _tc2_body,
        out_shape=jax.ShapeDtypeStruct((NP8, 128), jnp.float32),
    )(p1.reshape(NC, AP8, 128), m1p, dmatp, b1t, w2b)

    p2 = _agg_sc(m2p.reshape(N, H), src2d, dst2d)

    out = pl.pallas_call(
        _tc3_body,
        out_shape=jax.ShapeDtypeStruct((NP8, 8), jnp.float32),
    )(p2.reshape(NC, AP8, 128), m2p, dmatp, b2t, wls, bl.reshape(1, 1))

    return out.reshape(-1)


# speedup vs baseline: 1.5651x; 1.0812x over previous
import functools

import jax
import jax.numpy as jnp
from jax import lax
from jax.experimental import pallas as pl
from jax.experimental.pallas import tpu as pltpu
from jax.experimental.pallas import tpu_sc as plsc

N = 10000
F = 128
H = 16
E = 320000

NC, NS = 2, 16
NW = NC * NS
IDXW = 128
ROWS = E // IDXW
RPW = ROWS // NW
XT = ROWS - RPW * NW
KJ = 13
NSS = RPW // KJ
CH = KJ * IDXW
ACC_ROWS = 10240
RPT = ACC_ROWS // NS

_mesh = plsc.VectorSubcoreMesh(core_axis_name="c", subcore_axis_name="s")


@functools.partial(
    pl.kernel,
    mesh=_mesh,
    out_type=jax.ShapeDtypeStruct((NC, ACC_ROWS, H), jnp.float32),
    scratch_types=[
        pltpu.VMEM((RPW + 1, IDXW), jnp.int32),
        pltpu.VMEM((RPW + 1, IDXW), jnp.int32),
        pltpu.VMEM((CH, H), jnp.float32),
        pltpu.VMEM((CH, H), jnp.float32),
        pltpu.VMEM((RPT, H), jnp.float32),
        pltpu.VMEM_SHARED((ACC_ROWS, H), jnp.float32),
        pltpu.SemaphoreType.DMA,
        pltpu.SemaphoreType.DMA,
        pltpu.SemaphoreType.DMA,
    ],
    compiler_params=pltpu.CompilerParams(use_tc_tiling_on_sc=False),
)
def _agg_sc(m_hbm, src_hbm, dst_hbm, out_hbm, sidx, didx, rows0, rows1, zbuf, acc, sem0, sem1, ssem):
    c = lax.axis_index("c")
    s = lax.axis_index("s")
    wid = c * NS + s
    has_xtra = wid < XT
    start = wid * RPW + jnp.minimum(wid, XT)
    rowsb = (rows0, rows1)
    sems = (sem0, sem1)

    pltpu.sync_copy(src_hbm.at[pl.ds(start, RPW)], sidx.at[pl.ds(0, RPW)])
    pltpu.sync_copy(dst_hbm.at[pl.ds(start, RPW)], didx.at[pl.ds(0, RPW)])

    @pl.when(has_xtra)
    def _():
        pltpu.sync_copy(src_hbm.at[pl.ds(start + RPW, 1)], sidx.at[pl.ds(RPW, 1)])
        pltpu.sync_copy(dst_hbm.at[pl.ds(start + RPW, 1)], didx.at[pl.ds(RPW, 1)])

    def fire(ss):
        buf = rowsb[ss % 2]
        return [
            pltpu.async_copy(
                m_hbm.at[sidx.at[ss * KJ + j]],
                buf.at[pl.ds(j * IDXW, IDXW)],
                sems[ss % 2],
            )
            for j in range(KJ)
        ]

    pend = {0: fire(0), 1: fire(1)}

    def _z(i, carry):
        zbuf[i, :] = jnp.zeros((H,), jnp.float32)
        return carry

    lax.fori_loop(0, RPT, _z, 0)
    pltpu.sync_copy(zbuf, acc.at[pl.ds(s * RPT, RPT)])
    plsc.subcore_barrier()

    for ss in range(NSS):
        p = ss % 2
        for cp in pend.pop(ss):
            cp.wait()
        buf = rowsb[p]
        scs = [
            pltpu.async_copy(
                buf.at[pl.ds(j * IDXW, IDXW)],
                acc.at[didx.at[ss * KJ + j]],
                ssem,
                add=True,
            )
            for j in range(KJ)
        ]
        for cp in scs:
            cp.wait()
        if ss + 2 < NSS:
            pend[ss + 2] = fire(ss + 2)

    @pl.when(has_xtra)
    def _():
        pltpu.async_copy(
            m_hbm.at[sidx.at[RPW]], rows0.at[pl.ds(0, IDXW)], sem0
        ).wait()
        pltpu.sync_copy(rows0.at[pl.ds(0, IDXW)], acc.at[didx.at[RPW]], add=True)

    plsc.subcore_barrier()

    pltpu.sync_copy(acc.at[pl.ds(s * RPT, RPT)], zbuf)
    pltpu.sync_copy(zbuf, out_hbm.at[c].at[pl.ds(s * RPT, RPT)])


@functools.partial(
    pl.kernel,
    mesh=_mesh,
    out_type=jax.ShapeDtypeStruct((NC, ACC_ROWS, H), jnp.float32),
    scratch_types=[
        pltpu.VMEM((RPW + 1, IDXW), jnp.int32),
        pltpu.VMEM((RPT, H), jnp.float32),
        pltpu.VMEM_SHARED((ACC_ROWS, H), jnp.float32),
        pltpu.SemaphoreType.DMA,
    ],
    compiler_params=pltpu.CompilerParams(use_tc_tiling_on_sc=False),
)
def _deg_sc(dst_hbm, out_hbm, didx, rows, acc, dsem):
    c = lax.axis_index("c")
    s = lax.axis_index("s")
    wid = c * NS + s
    has_xtra = wid < XT
    start = wid * RPW + jnp.minimum(wid, XT)

    pltpu.sync_copy(dst_hbm.at[pl.ds(start, RPW)], didx.at[pl.ds(0, RPW)])

    @pl.when(has_xtra)
    def _():
        pltpu.sync_copy(dst_hbm.at[pl.ds(start + RPW, 1)], didx.at[pl.ds(RPW, 1)])

    def _z(i, carry):
        rows[i, :] = jnp.zeros((H,), jnp.float32)
        return carry

    lax.fori_loop(0, RPT, _z, 0)
    pltpu.sync_copy(rows, acc.at[pl.ds(s * RPT, RPT)])
    plsc.subcore_barrier()

    def _o(i, carry):
        rows[i, :] = jnp.ones((H,), jnp.float32)
        return carry

    lax.fori_loop(0, IDXW, _o, 0)

    for r0 in range(0, RPW, 26):
        scs = [
            pltpu.async_copy(
                rows.at[pl.ds(0, IDXW)], acc.at[didx.at[r0 + r]], dsem, add=True
            )
            for r in range(26)
        ]
        for cp in scs:
            cp.wait()

    @pl.when(has_xtra)
    def _():
        pltpu.async_copy(
            rows.at[pl.ds(0, IDXW)], acc.at[didx.at[RPW]], dsem, add=True
        ).wait()

    plsc.subcore_barrier()

    pltpu.sync_copy(acc.at[pl.ds(s * RPT, RPT)], rows)
    pltpu.sync_copy(rows, out_hbm.at[c].at[pl.ds(s * RPT, RPT)])


NP8 = N // 8
AP8 = ACC_ROWS // 8


def _tc1_body(degp_ref, x3_ref, w1_ref, m1p_ref, dmatp_ref):
    dd = degp_ref[0] + degp_ref[1] + 1.0
    dp = lax.rsqrt(dd)
    dmatp_ref[...] = dp
    for j in range(8):
        u1j = jnp.dot(
            x3_ref[:, j, :], w1_ref[...], preferred_element_type=jnp.float32
        )
        m1p_ref[:, j * H : (j + 1) * H] = dp[:NP8, j * H : (j + 1) * H] * u1j


def _tc2_body(p1_ref, m1p_ref, dmatp_ref, b1t_ref, w2b_ref, m2p_ref):
    dp = dmatp_ref[:NP8, :]
    h = dp * (p1_ref[0, :NP8, :] + p1_ref[1, :NP8, :] + m1p_ref[...]) + b1t_ref[...]
    h = jnp.maximum(h, 0.0)
    m2p_ref[...] = dp * jnp.dot(h, w2b_ref[...], preferred_element_type=jnp.float32)


def _tc3_body(p2_ref, m2p_ref, dmatp_ref, b2t_ref, wls_ref, bl_ref, out_ref):
    dp = dmatp_ref[:NP8, :]
    h = dp * (p2_ref[0, :NP8, :] + p2_ref[1, :NP8, :] + m2p_ref[...]) + b2t_ref[...]
    h = jnp.maximum(h, 0.0)
    out_ref[...] = (
        jnp.dot(h, wls_ref[...], preferred_element_type=jnp.float32) + bl_ref[...]
    )


def kernel(x, edge_index, W1, b1, W2, b2, Wl, bl):
    dst2d = edge_index[1].reshape(ROWS, IDXW)
    src_row = lax.optimization_barrier((edge_index[0], dst2d))[0]
    src2d = src_row.reshape(ROWS, IDXW)

    eye8 = jnp.eye(8, dtype=jnp.float32)
    w2b = jnp.kron(eye8, W2)
    wls = jnp.kron(eye8, Wl)
    b1t = jnp.tile(b1, 8).reshape(1, 128)
    b2t = jnp.tile(b2, 8).reshape(1, 128)

    degp = _deg_sc(dst2d)

    m1p, dmatp = pl.pallas_call(
        _tc1_body,
        out_shape=(
            jax.ShapeDtypeStruct((NP8, 128), jnp.float32),
            jax.ShapeDtypeStruct((AP8, 128), jnp.float32),
        ),
    )(degp.reshape(NC, AP8, 128), x.reshape(NP8, 8, F), W1)

    p1 = _agg_sc(m1p.reshape(N, H), src2d, dst2d)

    m2p = pl.pallas_call(
        _tc2_body,
        out_shape=jax.ShapeDtypeStruct((NP8, 128), jnp.float32),
    )(p1.reshape(NC, AP8, 128), m1p, dmatp, b1t, w2b)

    p2 = _agg_sc(m2p.reshape(N, H), src2d, dst2d)

    out = pl.pallas_call(
        _tc3_body,
        out_shape=jax.ShapeDtypeStruct((NP8, 8), jnp.float32),
    )(p2.reshape(NC, AP8, 128), m2p, dmatp, b2t, wls, bl.reshape(1, 1))

    return out.reshape(-1)


# speedup vs baseline: 1.6400x; 1.0479x over previous
import functools

import jax
import jax.numpy as jnp
from jax import lax
from jax.experimental import pallas as pl
from jax.experimental.pallas import tpu as pltpu
from jax.experimental.pallas import tpu_sc as plsc

N = 10000
F = 128
H = 16
E = 320000

NC, NS = 2, 16
NW = NC * NS
IDXW = 128
ROWS = E // IDXW
RPW = ROWS // NW
XT = ROWS - RPW * NW
KJ = 13
NSS = RPW // KJ
CH = KJ * IDXW
ACC_ROWS = 10240
RPT = ACC_ROWS // NS

_mesh = plsc.VectorSubcoreMesh(core_axis_name="c", subcore_axis_name="s")


@functools.partial(
    pl.kernel,
    mesh=_mesh,
    out_type=(
        jax.ShapeDtypeStruct((E,), jnp.int32),
        jax.ShapeDtypeStruct((E,), jnp.int32),
    ),
    scratch_types=[
        pltpu.VMEM(((RPW + 1) * IDXW,), jnp.int32),
    ],
    compiler_params=pltpu.CompilerParams(use_tc_tiling_on_sc=True),
)
def _cvt_sc(ei_hbm, src_out, dst_out, buf):
    c = lax.axis_index("c")
    s = lax.axis_index("s")
    wid = c * NS + s
    has_xtra = wid < XT
    start = (wid * RPW + jnp.minimum(wid, XT)) * IDXW

    for r, out in ((0, src_out), (1, dst_out)):
        pltpu.sync_copy(ei_hbm.at[r].at[pl.ds(start, RPW * IDXW)], buf.at[pl.ds(0, RPW * IDXW)])
        pltpu.sync_copy(buf.at[pl.ds(0, RPW * IDXW)], out.at[pl.ds(start, RPW * IDXW)])

        @pl.when(has_xtra)
        def _():
            pltpu.sync_copy(
                ei_hbm.at[r].at[pl.ds(start + RPW * IDXW, IDXW)],
                buf.at[pl.ds(0, IDXW)],
            )
            pltpu.sync_copy(
                buf.at[pl.ds(0, IDXW)], out.at[pl.ds(start + RPW * IDXW, IDXW)]
            )


@functools.partial(
    pl.kernel,
    mesh=_mesh,
    out_type=jax.ShapeDtypeStruct((NC, ACC_ROWS, H), jnp.float32),
    scratch_types=[
        pltpu.VMEM((RPW + 1, IDXW), jnp.int32),
        pltpu.VMEM((RPW + 1, IDXW), jnp.int32),
        pltpu.VMEM((CH, H), jnp.float32),
        pltpu.VMEM((CH, H), jnp.float32),
        pltpu.VMEM_SHARED((ACC_ROWS, H), jnp.float32),
        pltpu.SemaphoreType.DMA,
        pltpu.SemaphoreType.DMA,
        pltpu.SemaphoreType.DMA,
    ],
    compiler_params=pltpu.CompilerParams(use_tc_tiling_on_sc=False),
)
def _agg_sc(m_hbm, src_hbm, dst_hbm, out_hbm, sidx, didx, rows0, rows1, acc, sem0, sem1, ssem):
    c = lax.axis_index("c")
    s = lax.axis_index("s")
    wid = c * NS + s
    has_xtra = wid < XT
    start = wid * RPW + jnp.minimum(wid, XT)
    rowsb = (rows0, rows1)
    sems = (sem0, sem1)

    pltpu.sync_copy(src_hbm.at[pl.ds(start, RPW)], sidx.at[pl.ds(0, RPW)])
    pltpu.sync_copy(dst_hbm.at[pl.ds(start, RPW)], didx.at[pl.ds(0, RPW)])

    @pl.when(has_xtra)
    def _():
        pltpu.sync_copy(src_hbm.at[pl.ds(start + RPW, 1)], sidx.at[pl.ds(RPW, 1)])
        pltpu.sync_copy(dst_hbm.at[pl.ds(start + RPW, 1)], didx.at[pl.ds(RPW, 1)])

    def fire(ss):
        buf = rowsb[ss % 2]
        return [
            pltpu.async_copy(
                m_hbm.at[sidx.at[ss * KJ + j]],
                buf.at[pl.ds(j * IDXW, IDXW)],
                sems[ss % 2],
            )
            for j in range(KJ)
        ]

    pend = {0: fire(0)}

    def _z(i, carry):
        rows1[i, :] = jnp.zeros((H,), jnp.float32)
        return carry

    lax.fori_loop(0, RPT, _z, 0)
    pltpu.sync_copy(rows1.at[pl.ds(0, RPT)], acc.at[pl.ds(s * RPT, RPT)])
    pend[1] = fire(1)
    plsc.subcore_barrier()

    for ss in range(NSS):
        p = ss % 2
        for cp in pend.pop(ss):
            cp.wait()
        buf = rowsb[p]
        scs = [
            pltpu.async_copy(
                buf.at[pl.ds(j * IDXW, IDXW)],
                acc.at[didx.at[ss * KJ + j]],
                ssem,
                add=True,
            )
            for j in range(KJ)
        ]
        for cp in scs:
            cp.wait()
        if ss + 2 < NSS:
            pend[ss + 2] = fire(ss + 2)

    @pl.when(has_xtra)
    def _():
        pltpu.async_copy(
            m_hbm.at[sidx.at[RPW]], rows0.at[pl.ds(0, IDXW)], sem0
        ).wait()
        pltpu.sync_copy(rows0.at[pl.ds(0, IDXW)], acc.at[didx.at[RPW]], add=True)

    plsc.subcore_barrier()

    pltpu.sync_copy(acc.at[pl.ds(s * RPT, RPT)], rows0.at[pl.ds(0, RPT)])
    pltpu.sync_copy(rows0.at[pl.ds(0, RPT)], out_hbm.at[c].at[pl.ds(s * RPT, RPT)])


@functools.partial(
    pl.kernel,
    mesh=_mesh,
    out_type=jax.ShapeDtypeStruct((NC, ACC_ROWS, H), jnp.float32),
    scratch_types=[
        pltpu.VMEM((RPW + 1, IDXW), jnp.int32),
        pltpu.VMEM((RPT, H), jnp.float32),
        pltpu.VMEM_SHARED((ACC_ROWS, H), jnp.float32),
        pltpu.SemaphoreType.DMA,
    ],
    compiler_params=pltpu.CompilerParams(use_tc_tiling_on_sc=False),
)
def _deg_sc(dst_hbm, out_hbm, didx, rows, acc, dsem):
    c = lax.axis_index("c")
    s = lax.axis_index("s")
    wid = c * NS + s
    has_xtra = wid < XT
    start = wid * RPW + jnp.minimum(wid, XT)

    pltpu.sync_copy(dst_hbm.at[pl.ds(start, RPW)], didx.at[pl.ds(0, RPW)])

    @pl.when(has_xtra)
    def _():
        pltpu.sync_copy(dst_hbm.at[pl.ds(start + RPW, 1)], didx.at[pl.ds(RPW, 1)])

    def _z(i, carry):
        rows[i, :] = jnp.zeros((H,), jnp.float32)
        return carry

    lax.fori_loop(0, RPT, _z, 0)
    pltpu.sync_copy(rows, acc.at[pl.ds(s * RPT, RPT)])
    plsc.subcore_barrier()

    def _o(i, carry):
        rows[i, :] = jnp.ones((H,), jnp.float32)
        return carry

    lax.fori_loop(0, IDXW, _o, 0)

    for r0 in range(0, RPW, 26):
        scs = [
            pltpu.async_copy(
                rows.at[pl.ds(0, IDXW)], acc.at[didx.at[r0 + r]], dsem, add=True
            )
            for r in range(26)
        ]
        for cp in scs:
            cp.wait()

    @pl.when(has_xtra)
    def _():
        pltpu.async_copy(
            rows.at[pl.ds(0, IDXW)], acc.at[didx.at[RPW]], dsem, add=True
        ).wait()

    plsc.subcore_barrier()

    pltpu.sync_copy(acc.at[pl.ds(s * RPT, RPT)], rows)
    pltpu.sync_copy(rows, out_hbm.at[c].at[pl.ds(s * RPT, RPT)])


NP8 = N // 8
AP8 = ACC_ROWS // 8


def _tc1_body(degp_ref, x3_ref, w1_ref, m1p_ref, dmatp_ref):
    dd = degp_ref[0] + degp_ref[1] + 1.0
    dp = lax.rsqrt(dd)
    dmatp_ref[...] = dp
    for j in range(8):
        u1j = jnp.dot(
            x3_ref[:, j, :], w1_ref[...], preferred_element_type=jnp.float32
        )
        m1p_ref[:, j * H : (j + 1) * H] = dp[:NP8, j * H : (j + 1) * H] * u1j


def _tc2_body(p1_ref, m1p_ref, dmatp_ref, b1t_ref, w2b_ref, m2p_ref):
    dp = dmatp_ref[:NP8, :]
    h = dp * (p1_ref[0, :NP8, :] + p1_ref[1, :NP8, :] + m1p_ref[...]) + b1t_ref[...]
    h = jnp.maximum(h, 0.0)
    m2p_ref[...] = dp * jnp.dot(h, w2b_ref[...], preferred_element_type=jnp.float32)


def _tc3_body(p2_ref, m2p_ref, dmatp_ref, b2t_ref, wls_ref, bl_ref, out_ref):
    dp = dmatp_ref[:NP8, :]
    h = dp * (p2_ref[0, :NP8, :] + p2_ref[1, :NP8, :] + m2p_ref[...]) + b2t_ref[...]
    h = jnp.maximum(h, 0.0)
    out_ref[...] = (
        jnp.dot(h, wls_ref[...], preferred_element_type=jnp.float32) + bl_ref[...]
    )


def kernel(x, edge_index, W1, b1, W2, b2, Wl, bl):
    src_flat, dst_flat = _cvt_sc(edge_index)
    src2d = src_flat.reshape(ROWS, IDXW)
    dst2d = dst_flat.reshape(ROWS, IDXW)

    eye8 = jnp.eye(8, dtype=jnp.float32)
    w2b = jnp.kron(eye8, W2)
    wls = jnp.kron(eye8, Wl)
    b1t = jnp.tile(b1, 8).reshape(1, 128)
    b2t = jnp.tile(b2, 8).reshape(1, 128)

    degp = _deg_sc(dst2d)

    m1p, dmatp = pl.pallas_call(
        _tc1_body,
        out_shape=(
            jax.ShapeDtypeStruct((NP8, 128), jnp.float32),
            jax.ShapeDtypeStruct((AP8, 128), jnp.float32),
        ),
    )(degp.reshape(NC, AP8, 128), x.reshape(NP8, 8, F), W1)

    p1 = _agg_sc(m1p.reshape(N, H), src2d, dst2d)

    m2p = pl.pallas_call(
        _tc2_body,
        out_shape=jax.ShapeDtypeStruct((NP8, 128), jnp.float32),
    )(p1.reshape(NC, AP8, 128), m1p, dmatp, b1t, w2b)

    p2 = _agg_sc(m2p.reshape(N, H), src2d, dst2d)

    out = pl.pallas_call(
        _tc3_body,
        out_shape=jax.ShapeDtypeStruct((NP8, 8), jnp.float32),
    )(p2.reshape(NC, AP8, 128), m2p, dmatp, b2t, wls, bl.reshape(1, 1))

    return out.reshape(-1)


# speedup vs baseline: 1.7455x; 1.0643x over previous
import functools

import jax
import jax.numpy as jnp
from jax import lax
from jax.experimental import pallas as pl
from jax.experimental.pallas import tpu as pltpu
from jax.experimental.pallas import tpu_sc as plsc

N = 10000
F = 128
H = 16
E = 320000

NC, NS = 2, 16
NW = NC * NS
IDXW = 128
ROWS = E // IDXW
RPW = ROWS // NW
XT = ROWS - RPW * NW
KJ = 13
NSS = RPW // KJ
CH = KJ * IDXW
ACC_ROWS = 10240
RPT = ACC_ROWS // NS

_mesh = plsc.VectorSubcoreMesh(core_axis_name="c", subcore_axis_name="s")


@functools.partial(
    pl.kernel,
    mesh=_mesh,
    out_type=(
        jax.ShapeDtypeStruct((E,), jnp.int32),
        jax.ShapeDtypeStruct((E,), jnp.int32),
    ),
    scratch_types=[
        pltpu.VMEM(((RPW + 1) * IDXW,), jnp.int32),
    ],
    compiler_params=pltpu.CompilerParams(use_tc_tiling_on_sc=True),
)
def _cvt_sc(ei_hbm, src_out, dst_out, buf):
    c = lax.axis_index("c")
    s = lax.axis_index("s")
    wid = c * NS + s
    has_xtra = wid < XT
    start = (wid * RPW + jnp.minimum(wid, XT)) * IDXW

    for r, out in ((0, src_out), (1, dst_out)):
        pltpu.sync_copy(ei_hbm.at[r].at[pl.ds(start, RPW * IDXW)], buf.at[pl.ds(0, RPW * IDXW)])
        pltpu.sync_copy(buf.at[pl.ds(0, RPW * IDXW)], out.at[pl.ds(start, RPW * IDXW)])

        @pl.when(has_xtra)
        def _():
            pltpu.sync_copy(
                ei_hbm.at[r].at[pl.ds(start + RPW * IDXW, IDXW)],
                buf.at[pl.ds(0, IDXW)],
            )
            pltpu.sync_copy(
                buf.at[pl.ds(0, IDXW)], out.at[pl.ds(start + RPW * IDXW, IDXW)]
            )


@functools.partial(
    pl.kernel,
    mesh=_mesh,
    out_type=jax.ShapeDtypeStruct((NC, ACC_ROWS, H), jnp.float32),
    scratch_types=[
        pltpu.VMEM((RPW + 1, IDXW), jnp.int32),
        pltpu.VMEM((RPW + 1, IDXW), jnp.int32),
        pltpu.VMEM((CH, H), jnp.float32),
        pltpu.VMEM((CH, H), jnp.float32),
        pltpu.VMEM((CH, H), jnp.float32),
        pltpu.VMEM_SHARED((ACC_ROWS, H), jnp.float32),
        pltpu.SemaphoreType.DMA,
        pltpu.SemaphoreType.DMA,
        pltpu.SemaphoreType.DMA,
        pltpu.SemaphoreType.DMA,
    ],
    compiler_params=pltpu.CompilerParams(use_tc_tiling_on_sc=False),
)
def _agg_sc(m_hbm, src_hbm, dst_hbm, out_hbm, sidx, didx, rows0, rows1, rows2, acc, sem0, sem1, sem2, ssem):
    c = lax.axis_index("c")
    s = lax.axis_index("s")
    wid = c * NS + s
    has_xtra = wid < XT
    start = wid * RPW + jnp.minimum(wid, XT)
    rowsb = (rows0, rows1, rows2)
    sems = (sem0, sem1, sem2)
    NB = 3

    pltpu.sync_copy(src_hbm.at[pl.ds(start, RPW)], sidx.at[pl.ds(0, RPW)])
    pltpu.sync_copy(dst_hbm.at[pl.ds(start, RPW)], didx.at[pl.ds(0, RPW)])

    @pl.when(has_xtra)
    def _():
        pltpu.sync_copy(src_hbm.at[pl.ds(start + RPW, 1)], sidx.at[pl.ds(RPW, 1)])
        pltpu.sync_copy(dst_hbm.at[pl.ds(start + RPW, 1)], didx.at[pl.ds(RPW, 1)])

    def fire(ss):
        buf = rowsb[ss % NB]
        return [
            pltpu.async_copy(
                m_hbm.at[sidx.at[ss * KJ + j]],
                buf.at[pl.ds(j * IDXW, IDXW)],
                sems[ss % NB],
            )
            for j in range(KJ)
        ]

    pend = {0: fire(0), 1: fire(1)}

    def _z(i, carry):
        rows2[i, :] = jnp.zeros((H,), jnp.float32)
        return carry

    lax.fori_loop(0, RPT, _z, 0)
    pltpu.sync_copy(rows2.at[pl.ds(0, RPT)], acc.at[pl.ds(s * RPT, RPT)])
    pend[2] = fire(2)
    plsc.subcore_barrier()

    for ss in range(NSS):
        p = ss % NB
        for cp in pend.pop(ss):
            cp.wait()
        buf = rowsb[p]
        scs = [
            pltpu.async_copy(
                buf.at[pl.ds(j * IDXW, IDXW)],
                acc.at[didx.at[ss * KJ + j]],
                ssem,
                add=True,
            )
            for j in range(KJ)
        ]
        for cp in scs:
            cp.wait()
        if ss + NB < NSS:
            pend[ss + NB] = fire(ss + NB)

    @pl.when(has_xtra)
    def _():
        pltpu.async_copy(
            m_hbm.at[sidx.at[RPW]], rows0.at[pl.ds(0, IDXW)], sem0
        ).wait()
        pltpu.sync_copy(rows0.at[pl.ds(0, IDXW)], acc.at[didx.at[RPW]], add=True)

    plsc.subcore_barrier()

    pltpu.sync_copy(acc.at[pl.ds(s * RPT, RPT)], rows0.at[pl.ds(0, RPT)])
    pltpu.sync_copy(rows0.at[pl.ds(0, RPT)], out_hbm.at[c].at[pl.ds(s * RPT, RPT)])


@functools.partial(
    pl.kernel,
    mesh=_mesh,
    out_type=jax.ShapeDtypeStruct((NC, ACC_ROWS, H), jnp.float32),
    scratch_types=[
        pltpu.VMEM((RPW + 1, IDXW), jnp.int32),
        pltpu.VMEM((RPT, H), jnp.float32),
        pltpu.VMEM_SHARED((ACC_ROWS, H), jnp.float32),
        pltpu.SemaphoreType.DMA,
    ],
    compiler_params=pltpu.CompilerParams(use_tc_tiling_on_sc=False),
)
def _deg_sc(dst_hbm, out_hbm, didx, rows, acc, dsem):
    c = lax.axis_index("c")
    s = lax.axis_index("s")
    wid = c * NS + s
    has_xtra = wid < XT
    start = wid * RPW + jnp.minimum(wid, XT)

    pltpu.sync_copy(dst_hbm.at[pl.ds(start, RPW)], didx.at[pl.ds(0, RPW)])

    @pl.when(has_xtra)
    def _():
        pltpu.sync_copy(dst_hbm.at[pl.ds(start + RPW, 1)], didx.at[pl.ds(RPW, 1)])

    def _z(i, carry):
        rows[i, :] = jnp.zeros((H,), jnp.float32)
        return carry

    lax.fori_loop(0, RPT, _z, 0)
    pltpu.sync_copy(rows, acc.at[pl.ds(s * RPT, RPT)])
    plsc.subcore_barrier()

    def _o(i, carry):
        rows[i, :] = jnp.ones((H,), jnp.float32)
        return carry

    lax.fori_loop(0, IDXW, _o, 0)

    for r0 in range(0, RPW, 26):
        scs = [
            pltpu.async_copy(
                rows.at[pl.ds(0, IDXW)], acc.at[didx.at[r0 + r]], dsem, add=True
            )
            for r in range(26)
        ]
        for cp in scs:
            cp.wait()

    @pl.when(has_xtra)
    def _():
        pltpu.async_copy(
            rows.at[pl.ds(0, IDXW)], acc.at[didx.at[RPW]], dsem, add=True
        ).wait()

    plsc.subcore_barrier()

    pltpu.sync_copy(acc.at[pl.ds(s * RPT, RPT)], rows)
    pltpu.sync_copy(rows, out_hbm.at[c].at[pl.ds(s * RPT, RPT)])


NP8 = N // 8
AP8 = ACC_ROWS // 8


def _tc0_body(x3_ref, w1_ref, u1p_ref):
    for j in range(8):
        u1p_ref[:, j * H : (j + 1) * H] = jnp.dot(
            x3_ref[:, j, :], w1_ref[...], preferred_element_type=jnp.float32
        )


def _tc1_body(degp_ref, u1p_ref, m1p_ref, dmatp_ref):
    dd = degp_ref[0] + degp_ref[1] + 1.0
    dp = lax.rsqrt(dd)
    dmatp_ref[...] = dp
    m1p_ref[...] = dp[:NP8, :] * u1p_ref[...]


def _tc2_body(p1_ref, m1p_ref, dmatp_ref, b1t_ref, w2b_ref, m2p_ref):
    dp = dmatp_ref[:NP8, :]
    h = dp * (p1_ref[0, :NP8, :] + p1_ref[1, :NP8, :] + m1p_ref[...]) + b1t_ref[...]
    h = jnp.maximum(h, 0.0)
    m2p_ref[...] = dp * jnp.dot(h, w2b_ref[...], preferred_element_type=jnp.float32)


def _tc3_body(p2_ref, m2p_ref, dmatp_ref, b2t_ref, wls_ref, bl_ref, out_ref):
    dp = dmatp_ref[:NP8, :]
    h = dp * (p2_ref[0, :NP8, :] + p2_ref[1, :NP8, :] + m2p_ref[...]) + b2t_ref[...]
    h = jnp.maximum(h, 0.0)
    out_ref[...] = (
        jnp.dot(h, wls_ref[...], preferred_element_type=jnp.float32) + bl_ref[...]
    )


def kernel(x, edge_index, W1, b1, W2, b2, Wl, bl):
    src_flat, dst_flat = _cvt_sc(edge_index)
    src2d = src_flat.reshape(ROWS, IDXW)
    dst2d = dst_flat.reshape(ROWS, IDXW)

    eye8 = jnp.eye(8, dtype=jnp.float32)
    w2b = jnp.kron(eye8, W2)
    wls = jnp.kron(eye8, Wl)
    b1t = jnp.tile(b1, 8).reshape(1, 128)
    b2t = jnp.tile(b2, 8).reshape(1, 128)

    degp = _deg_sc(dst2d)

    u1p = pl.pallas_call(
        _tc0_body,
        out_shape=jax.ShapeDtypeStruct((NP8, 128), jnp.float32),
    )(x.reshape(NP8, 8, F), W1)

    m1p, dmatp = pl.pallas_call(
        _tc1_body,
        out_shape=(
            jax.ShapeDtypeStruct((NP8, 128), jnp.float32),
            jax.ShapeDtypeStruct((AP8, 128), jnp.float32),
        ),
    )(degp.reshape(NC, AP8, 128), u1p)

    p1 = _agg_sc(m1p.reshape(N, H), src2d, dst2d)

    m2p = pl.pallas_call(
        _tc2_body,
        out_shape=jax.ShapeDtypeStruct((NP8, 128), jnp.float32),
    )(p1.reshape(NC, AP8, 128), m1p, dmatp, b1t, w2b)

    p2 = _agg_sc(m2p.reshape(N, H), src2d, dst2d)

    out = pl.pallas_call(
        _tc3_body,
        out_shape=jax.ShapeDtypeStruct((NP8, 8), jnp.float32),
    )(p2.reshape(NC, AP8, 128), m2p, dmatp, b2t, wls, bl.reshape(1, 1))

    return out.reshape(-1)
